# Initial kernel scaffold; baseline (speedup 1.0000x reference)
import jax
import jax.numpy as jnp
from jax.experimental import pallas as pl


def kernel(rel_rep, fb_key, fb_value, rel_embed_w, wpred_w1, wpred_b1, wpred_w2, wpred_b2, attn_q_w, attn_q_b, attn_k_w, attn_k_b, attn_v_w, attn_v_b):
    raise NotImplementedError("write your pallas kernel here")


# speedup vs baseline: 1.2099x; 1.2099x over previous
import functools
import jax
import jax.numpy as jnp
from jax import lax
from jax.experimental import pallas as pl
from jax.experimental.pallas import tpu as pltpu

Q = 256
M = 16384
D = 2048
K = 8
NREL = 51
BLK = 128
NBLK = M // BLK
MBLK = 1024
HI = jax.lax.Precision.DEFAULT


def _sim_body(rel_ref, fb_ref, simb_ref, bmax_ref):
    x = rel_ref[...]
    qn = x / (jnp.sqrt(jnp.sum(x * x, axis=-1, keepdims=True)) + 1e-12)
    f = fb_ref[...]
    fn = f / (jnp.sqrt(jnp.sum(f * f, axis=-1, keepdims=True)) + 1e-12)
    s = lax.dot_general(qn, fn, (((1,), (1,)), ((), ())),
                        preferred_element_type=jnp.float32, precision=HI)
    sb = s.reshape(Q, MBLK // BLK, BLK).transpose(1, 0, 2)
    simb_ref[...] = sb.reshape(MBLK // BLK * Q, BLK)
    bmax_ref[...] = jnp.max(sb, axis=2)


def _sim_blocked(rel_rep, fb_key):
    nsteps = M // MBLK
    return pl.pallas_call(
        _sim_body,
        grid=(nsteps,),
        in_specs=[
            pl.BlockSpec((Q, D), lambda i: (0, 0)),
            pl.BlockSpec((MBLK, D), lambda i: (i, 0)),
        ],
        out_specs=[
            pl.BlockSpec((MBLK // BLK * Q, BLK), lambda i: (i, 0)),
            pl.BlockSpec((MBLK // BLK, Q), lambda i: (i, 0)),
        ],
        out_shape=[
            jax.ShapeDtypeStruct((NBLK * Q, BLK), jnp.float32),
            jax.ShapeDtypeStruct((NBLK, Q), jnp.float32),
        ],
    )(rel_rep, fb_key)


def _bmaxT_body(bm_ref, out_ref):
    out_ref[...] = bm_ref[...].T


def _bmaxT(bmax_bq):
    return pl.pallas_call(
        _bmaxT_body,
        out_shape=jax.ShapeDtypeStruct((Q, NBLK), jnp.float32),
    )(bmax_bq)


def _matmul_body(x_ref, w_ref, b_ref, o_ref, *, trans):
    dn = (((1,), (1,)), ((), ())) if trans else (((1,), (0,)), ((), ()))
    o_ref[...] = lax.dot_general(
        x_ref[...], w_ref[...], dn,
        preferred_element_type=jnp.float32, precision=HI) + b_ref[...]


def _proj(x, w, b, *, trans=False, nblk=16):
    n = w.shape[0] if trans else w.shape[1]
    bn = n // nblk
    w_spec = (pl.BlockSpec((bn, w.shape[1]), lambda i: (i, 0)) if trans
              else pl.BlockSpec((w.shape[0], bn), lambda i: (0, i)))
    return pl.pallas_call(
        functools.partial(_matmul_body, trans=trans),
        grid=(nblk,),
        in_specs=[
            pl.BlockSpec(x.shape, lambda i: (0, 0)),
            w_spec,
            pl.BlockSpec((bn,), lambda i: (i,)),
        ],
        out_specs=pl.BlockSpec((x.shape[0], bn), lambda i: (0, i)),
        out_shape=jax.ShapeDtypeStruct((x.shape[0], n), jnp.float32),
    )(x, w, b)


def _pred_table_body(e_ref, w1_ref, b1_ref, w2_ref, b2_ref, o_ref):
    h = jnp.maximum(
        lax.dot_general(e_ref[...], w1_ref[...], (((1,), (0,)), ((), ())),
                        preferred_element_type=jnp.float32, precision=HI)
        + b1_ref[...], 0.0)
    o_ref[...] = lax.dot_general(h, w2_ref[...], (((1,), (0,)), ((), ())),
                                 preferred_element_type=jnp.float32,
                                 precision=HI) + b2_ref[...]


def _pred_table(rel_embed_w, w1, b1, w2, b2):
    return pl.pallas_call(
        _pred_table_body,
        out_shape=jax.ShapeDtypeStruct((NREL, D), jnp.float32),
    )(rel_embed_w, w1, b1[None, :], w2, b2[None, :])


def _final_body(wk_ref, wp_ref, vtl_ref, vbl_ref, bv_ref, rr_ref, o_ref):
    fx = (lax.dot_general(wk_ref[...], vtl_ref[...], (((1,), (0,)), ((), ())),
                          preferred_element_type=jnp.float32, precision=HI)
          + lax.dot_general(wp_ref[...], vbl_ref[...], (((1,), (0,)), ((), ())),
                            preferred_element_type=jnp.float32, precision=HI)
          + bv_ref[...])
    rr = rr_ref[...]
    o_ref[...] = jnp.maximum(fx + rr, 0.0) - (fx - rr) ** 2


def _final(wk, wp, vtl, vbl, bvl, rel_rep, nblk=16):
    bn = D // nblk
    return pl.pallas_call(
        _final_body,
        grid=(nblk,),
        in_specs=[
            pl.BlockSpec((Q, D), lambda i: (0, 0)),
            pl.BlockSpec((Q, D), lambda i: (0, 0)),
            pl.BlockSpec((D, bn), lambda i: (0, i)),
            pl.BlockSpec((D, bn), lambda i: (0, i)),
            pl.BlockSpec((bn,), lambda i: (i,)),
            pl.BlockSpec((Q, bn), lambda i: (0, i)),
        ],
        out_specs=pl.BlockSpec((Q, bn), lambda i: (0, i)),
        out_shape=jax.ShapeDtypeStruct((Q, D), jnp.float32),
    )(wk, wp, vtl, vbl, bvl, rel_rep)


def kernel(rel_rep, fb_key, fb_value, rel_embed_w, wpred_w1, wpred_b1,
           wpred_w2, wpred_b2, attn_q_w, attn_q_b, attn_k_w, attn_k_b,
           attn_v_w, attn_v_b):
    simb, bmax_bq = _sim_blocked(rel_rep, fb_key)
    bmax = _bmaxT(bmax_bq)
    del bmax

    sim = simb.reshape(NBLK, Q, BLK).transpose(1, 0, 2).reshape(Q, M)
    topk_sim, topk_idx = lax.top_k(sim, K)
    ret_keys = jnp.take(fb_key, topk_idx, axis=0)
    ret_vals = jnp.take(fb_value, topk_idx, axis=0)

    q = _proj(rel_rep, attn_q_w, attn_q_b)
    t = _proj(q, attn_k_w, jnp.zeros((D,), jnp.float32), trans=True)
    ptab = _pred_table(rel_embed_w, wpred_w1, wpred_b1, wpred_w2, wpred_b2)
    pred_emb = jnp.take(ptab, ret_vals, axis=0)

    scores = jnp.einsum('qkd,qd->qk', ret_keys, t) / jnp.sqrt(float(D))
    attn = jax.nn.softmax(scores, axis=-1)
    wk = jnp.einsum('qk,qkd->qd', attn, ret_keys)
    wp = jnp.einsum('qk,qkd->qd', attn, pred_emb)

    out = _final(wk, wp, attn_v_w[:D, :D], attn_v_w[D:, :D], attn_v_b[:D],
                 rel_rep)
    return out, topk_sim, ret_vals


# speedup vs baseline: 2.2084x; 1.8252x over previous
import functools
import jax
import jax.numpy as jnp
from jax import lax
from jax.experimental import pallas as pl
from jax.experimental.pallas import tpu as pltpu
from jax.experimental.pallas import tpu_sc as plsc

Q = 256
M = 16384
D = 2048
K = 8
NREL = 51
BLK = 128
NBLK = M // BLK
MBLK = 1024
HI = jax.lax.Precision.DEFAULT


def _sim_body(rel_ref, fb_ref, simb_ref, bmax_ref):
    x = rel_ref[...]
    qn = x / (jnp.sqrt(jnp.sum(x * x, axis=-1, keepdims=True)) + 1e-12)
    f = fb_ref[...]
    fn = f / (jnp.sqrt(jnp.sum(f * f, axis=-1, keepdims=True)) + 1e-12)
    s = lax.dot_general(qn, fn, (((1,), (1,)), ((), ())),
                        preferred_element_type=jnp.float32, precision=HI)
    sb = s.reshape(Q, MBLK // BLK, BLK).transpose(1, 0, 2)
    simb_ref[...] = sb.reshape(MBLK // BLK * Q, BLK)
    bmax_ref[...] = jnp.max(sb, axis=2)


def _sim_blocked(rel_rep, fb_key):
    nsteps = M // MBLK
    return pl.pallas_call(
        _sim_body,
        grid=(nsteps,),
        in_specs=[
            pl.BlockSpec((Q, D), lambda i: (0, 0)),
            pl.BlockSpec((MBLK, D), lambda i: (i, 0)),
        ],
        out_specs=[
            pl.BlockSpec((MBLK // BLK * Q, BLK), lambda i: (i, 0)),
            pl.BlockSpec((MBLK // BLK, Q), lambda i: (i, 0)),
        ],
        out_shape=[
            jax.ShapeDtypeStruct((NBLK * Q, BLK), jnp.float32),
            jax.ShapeDtypeStruct((NBLK, Q), jnp.float32),
        ],
    )(rel_rep, fb_key)


def _bmaxT_body(bm_ref, out_ref):
    out_ref[...] = bm_ref[...].T


def _bmaxT(bmax_bq):
    return pl.pallas_call(
        _bmaxT_body,
        out_shape=jax.ShapeDtypeStruct((Q, NBLK), jnp.float32),
    )(bmax_bq)


def _matmul_body(x_ref, w_ref, b_ref, o_ref, *, trans):
    dn = (((1,), (1,)), ((), ())) if trans else (((1,), (0,)), ((), ()))
    o_ref[...] = lax.dot_general(
        x_ref[...], w_ref[...], dn,
        preferred_element_type=jnp.float32, precision=HI) + b_ref[...]


def _proj(x, w, b, *, trans=False, nblk=16):
    n = w.shape[0] if trans else w.shape[1]
    bn = n // nblk
    w_spec = (pl.BlockSpec((bn, w.shape[1]), lambda i: (i, 0)) if trans
              else pl.BlockSpec((w.shape[0], bn), lambda i: (0, i)))
    return pl.pallas_call(
        functools.partial(_matmul_body, trans=trans),
        grid=(nblk,),
        in_specs=[
            pl.BlockSpec(x.shape, lambda i: (0, 0)),
            w_spec,
            pl.BlockSpec((bn,), lambda i: (i,)),
        ],
        out_specs=pl.BlockSpec((x.shape[0], bn), lambda i: (0, i)),
        out_shape=jax.ShapeDtypeStruct((x.shape[0], n), jnp.float32),
    )(x, w, b)


def _pred_table_body(e_ref, w1_ref, b1_ref, w2_ref, b2_ref, o_ref):
    h = jnp.maximum(
        lax.dot_general(e_ref[...], w1_ref[...], (((1,), (0,)), ((), ())),
                        preferred_element_type=jnp.float32, precision=HI)
        + b1_ref[...], 0.0)
    o_ref[...] = lax.dot_general(h, w2_ref[...], (((1,), (0,)), ((), ())),
                                 preferred_element_type=jnp.float32,
                                 precision=HI) + b2_ref[...]


def _pred_table(rel_embed_w, w1, b1, w2, b2):
    return pl.pallas_call(
        _pred_table_body,
        out_shape=jax.ShapeDtypeStruct((NREL, D), jnp.float32),
    )(rel_embed_w, w1, b1[None, :], w2, b2[None, :])


def _final_body(wk_ref, wp_ref, vtl_ref, vbl_ref, bv_ref, rr_ref, o_ref):
    fx = (lax.dot_general(wk_ref[...], vtl_ref[...], (((1,), (0,)), ((), ())),
                          preferred_element_type=jnp.float32, precision=HI)
          + lax.dot_general(wp_ref[...], vbl_ref[...], (((1,), (0,)), ((), ())),
                            preferred_element_type=jnp.float32, precision=HI)
          + bv_ref[...])
    rr = rr_ref[...]
    o_ref[...] = jnp.maximum(fx + rr, 0.0) - (fx - rr) ** 2


def _final(wk, wp, vtl, vbl, bvl, rel_rep, nblk=16):
    bn = D // nblk
    return pl.pallas_call(
        _final_body,
        grid=(nblk,),
        in_specs=[
            pl.BlockSpec((Q, D), lambda i: (0, 0)),
            pl.BlockSpec((Q, D), lambda i: (0, 0)),
            pl.BlockSpec((D, bn), lambda i: (0, i)),
            pl.BlockSpec((D, bn), lambda i: (0, i)),
            pl.BlockSpec((bn,), lambda i: (i,)),
            pl.BlockSpec((Q, bn), lambda i: (0, i)),
        ],
        out_specs=pl.BlockSpec((Q, bn), lambda i: (0, i)),
        out_shape=jax.ShapeDtypeStruct((Q, D), jnp.float32),
    )(wk, wp, vtl, vbl, bvl, rel_rep)


_NW = 32
_QPW = Q // _NW
_NCH = BLK // 16
_NEG = float("-inf")
_SCALE = 1.0 / (float(D) ** 0.5)


def _lanes():
    return lax.iota(jnp.int32, 16)


def _merge_top16(pool_v, pool_i, ch, ids):
    c_desc, ci_desc = plsc.sort_key_val(ch, ids, descending=True)
    take = c_desc > pool_v
    nv = jnp.where(take, c_desc, pool_v)
    ni = jnp.where(take, ci_desc, pool_i)
    return plsc.sort_key_val(nv, ni, descending=False)


def _sc_retrieve_body(simb_hbm, bmax_hbm, fbk_hbm, fbv_hbm, ptab_hbm, t_hbm,
                      osim_hbm, ovals_hbm, wk_hbm, wp_hbm,
                      fbval_v, bm_v, surv_v, sidx_v, blk_v, idx8_v, vals8_v,
                      rows_v, prow_v, t_v, wk_v, wp_v, osim_v, ovals_v,
                      sem0, sem1):
    wid = lax.axis_index("s") * 2 + lax.axis_index("c")
    pltpu.sync_copy(fbv_hbm, fbval_v)
    lanes = _lanes()

    def per_query(j, _):
        qg = wid * _QPW + j
        pltpu.sync_copy(bmax_hbm.at[qg], bm_v)

        pool_v = jnp.full((16,), _NEG, jnp.float32)
        pool_i = jnp.zeros((16,), jnp.int32)
        for c in range(_NCH):
            ch = bm_v[pl.ds(c * 16, 16)]
            pool_v, pool_i = _merge_top16(pool_v, pool_i, ch, lanes)
        m8 = jnp.max(jnp.where(lanes == 8, pool_v, _NEG))

        surv_v[pl.ds(0, 16)] = jnp.zeros((16,), jnp.int32)
        cnt = jnp.int32(0)
        for c in range(_NCH):
            ch = bm_v[pl.ds(c * 16, 16)]
            msk = ch >= m8
            pos = cnt + plsc.cumsum(msk.astype(jnp.int32)) - 1
            plsc.store_scatter(surv_v, [pos], lanes + c * 16, mask=msk)
            cnt = cnt + jnp.sum(msk.astype(jnp.int32))
        k = jnp.minimum(cnt, 16)

        sidx_v[...] = surv_v[pl.ds(0, 16)] * Q + qg
        pltpu.async_copy(simb_hbm.at[sidx_v], blk_v, sem0).wait()

        def scan_block(s, carry):
            pool_v, pool_i, thresh = carry
            sb = surv_v[pl.ds(s, 16)][0]
            for c in range(_NCH):
                ch = blk_v[s, pl.ds(c * 16, 16)]
                ids = lanes + (sb * BLK + c * 16)

                def do_merge(args):
                    pv, pi = args
                    pv, pi = _merge_top16(pv, pi, ch, ids)
                    return pv, pi

                pool_v, pool_i = lax.cond(
                    jnp.any(ch > thresh), do_merge, lambda a: a,
                    (pool_v, pool_i))
                thresh = jnp.min(pool_v)
            return pool_v, pool_i, thresh

        pool_v = jnp.full((16,), _NEG, jnp.float32)
        pool_i = jnp.zeros((16,), jnp.int32)
        pool_v, pool_i, _ = lax.fori_loop(
            0, k, scan_block, (pool_v, pool_i, jnp.float32(_NEG)))

        top_v, top_i = plsc.sort_key_val(pool_v, pool_i, descending=True)
        low8 = lanes < 8
        plsc.store_scatter(osim_v, [j * K + lanes], top_v, mask=low8)
        plsc.store_scatter(idx8_v, [lanes], top_i, mask=low8)
        rv = plsc.load_gather(fbval_v, [top_i])
        plsc.store_scatter(ovals_v, [j * K + lanes], rv, mask=low8)
        plsc.store_scatter(vals8_v, [lanes], rv, mask=low8)

        h1 = pltpu.async_copy(fbk_hbm.at[idx8_v], rows_v, sem0)
        h2 = pltpu.async_copy(ptab_hbm.at[vals8_v], prow_v, sem1)
        pltpu.sync_copy(t_hbm.at[qg], t_v)
        h1.wait()
        h2.wait()

        def dot_chunk(i, accs):
            tch = t_v[pl.ds(i * 16, 16)]
            return tuple(accs[kk] + rows_v[kk, pl.ds(i * 16, 16)] * tch
                         for kk in range(K))

        accs = lax.fori_loop(0, D // 16, dot_chunk,
                             tuple(jnp.zeros((16,), jnp.float32)
                                   for _ in range(K)))
        svec = jnp.full((16,), _NEG, jnp.float32)
        for kk in range(K):
            svec = jnp.where(lanes == kk, jnp.sum(accs[kk]), svec)
        mx = jnp.max(svec)
        e = jnp.exp((svec - mx) * _SCALE)
        attn = e / jnp.sum(e)
        aw = [jnp.sum(jnp.where(lanes == kk, attn, 0.0)) for kk in range(K)]

        def wsum_chunk(i, _):
            sl = pl.ds(i * 16, 16)
            acc_k = aw[0] * rows_v[0, sl]
            acc_p = aw[0] * prow_v[0, sl]
            for kk in range(1, K):
                acc_k = acc_k + aw[kk] * rows_v[kk, sl]
                acc_p = acc_p + aw[kk] * prow_v[kk, sl]
            wk_v[sl] = acc_k
            wp_v[sl] = acc_p
            return 0

        lax.fori_loop(0, D // 16, wsum_chunk, 0)
        pltpu.sync_copy(wk_v, wk_hbm.at[qg])
        pltpu.sync_copy(wp_v, wp_hbm.at[qg])
        return 0

    lax.fori_loop(0, _QPW, per_query, 0)
    pltpu.sync_copy(osim_v, osim_hbm.at[pl.ds(wid * _QPW * K, _QPW * K)])
    pltpu.sync_copy(ovals_v, ovals_hbm.at[pl.ds(wid * _QPW * K, _QPW * K)])


def _sc_retrieve(simb, bmax, fb_key, fb_value, ptab, t):
    mesh = plsc.VectorSubcoreMesh(core_axis_name="c", subcore_axis_name="s")
    f = pl.kernel(
        _sc_retrieve_body,
        out_type=[
            jax.ShapeDtypeStruct((Q * K,), jnp.float32),
            jax.ShapeDtypeStruct((Q * K,), jnp.int32),
            jax.ShapeDtypeStruct((Q, D), jnp.float32),
            jax.ShapeDtypeStruct((Q, D), jnp.float32),
        ],
        mesh=mesh,
        compiler_params=pltpu.CompilerParams(needs_layout_passes=False),
        scratch_types=[
            pltpu.VMEM((M,), jnp.int32),
            pltpu.VMEM((NBLK,), jnp.float32),
            pltpu.VMEM((NBLK + 16,), jnp.int32),
            pltpu.VMEM((16,), jnp.int32),
            pltpu.VMEM((16, BLK), jnp.float32),
            pltpu.VMEM((K,), jnp.int32),
            pltpu.VMEM((K,), jnp.int32),
            pltpu.VMEM((K, D), jnp.float32),
            pltpu.VMEM((K, D), jnp.float32),
            pltpu.VMEM((D,), jnp.float32),
            pltpu.VMEM((D,), jnp.float32),
            pltpu.VMEM((D,), jnp.float32),
            pltpu.VMEM((_QPW * K,), jnp.float32),
            pltpu.VMEM((_QPW * K,), jnp.int32),
            pltpu.SemaphoreType.DMA,
            pltpu.SemaphoreType.DMA,
        ],
    )
    return f(simb, bmax, fb_key, fb_value, ptab, t)


def kernel(rel_rep, fb_key, fb_value, rel_embed_w, wpred_w1, wpred_b1,
           wpred_w2, wpred_b2, attn_q_w, attn_q_b, attn_k_w, attn_k_b,
           attn_v_w, attn_v_b):
    q = _proj(rel_rep, attn_q_w, attn_q_b)
    t = _proj(q, attn_k_w, jnp.zeros((D,), jnp.float32), trans=True)
    ptab = _pred_table(rel_embed_w, wpred_w1, wpred_b1, wpred_w2, wpred_b2)

    simb, bmax_bq = _sim_blocked(rel_rep, fb_key)
    bmax = _bmaxT(bmax_bq)

    topk_sim_f, ret_vals_f, wk, wp = _sc_retrieve(
        simb, bmax, fb_key, fb_value, ptab, t)
    topk_sim = topk_sim_f.reshape(Q, K)
    ret_vals = ret_vals_f.reshape(Q, K)

    out = _final(wk, wp, attn_v_w[:D, :D], attn_v_w[D:, :D], attn_v_b[:D],
                 rel_rep)
    return out, topk_sim, ret_vals


# speedup vs baseline: 2.3317x; 1.0559x over previous
import functools
import jax
import jax.numpy as jnp
from jax import lax
from jax.experimental import pallas as pl
from jax.experimental.pallas import tpu as pltpu
from jax.experimental.pallas import tpu_sc as plsc

Q = 256
M = 16384
D = 2048
K = 8
NREL = 51
BLK = 128
NBLK = M // BLK
MBLK = 1024
HI = jax.lax.Precision.DEFAULT


def _sim_body(rel_ref, fb_ref, simb_ref, bmax_ref):
    x = rel_ref[...]
    qn = x / (jnp.sqrt(jnp.sum(x * x, axis=-1, keepdims=True)) + 1e-12)
    f = fb_ref[...]
    fn = f / (jnp.sqrt(jnp.sum(f * f, axis=-1, keepdims=True)) + 1e-12)
    s = lax.dot_general(qn, fn, (((1,), (1,)), ((), ())),
                        preferred_element_type=jnp.float32, precision=HI)
    sb = s.reshape(Q, MBLK // BLK, BLK).transpose(1, 0, 2)
    simb_ref[...] = sb.reshape(MBLK // BLK * Q, BLK)
    bmax_ref[...] = jnp.max(sb, axis=2)


def _sim_blocked(rel_rep, fb_key):
    nsteps = M // MBLK
    return pl.pallas_call(
        _sim_body,
        grid=(nsteps,),
        in_specs=[
            pl.BlockSpec((Q, D), lambda i: (0, 0)),
            pl.BlockSpec((MBLK, D), lambda i: (i, 0)),
        ],
        out_specs=[
            pl.BlockSpec((MBLK // BLK * Q, BLK), lambda i: (i, 0)),
            pl.BlockSpec((MBLK // BLK, Q), lambda i: (i, 0)),
        ],
        out_shape=[
            jax.ShapeDtypeStruct((NBLK * Q, BLK), jnp.float32),
            jax.ShapeDtypeStruct((NBLK, Q), jnp.float32),
        ],
    )(rel_rep, fb_key)


def _bmaxT_body(bm_ref, out_ref):
    out_ref[...] = bm_ref[...].T


def _bmaxT(bmax_bq):
    return pl.pallas_call(
        _bmaxT_body,
        out_shape=jax.ShapeDtypeStruct((Q, NBLK), jnp.float32),
    )(bmax_bq)


def _matmul_body(x_ref, w_ref, b_ref, o_ref, *, trans):
    dn = (((1,), (1,)), ((), ())) if trans else (((1,), (0,)), ((), ()))
    o_ref[...] = lax.dot_general(
        x_ref[...], w_ref[...], dn,
        preferred_element_type=jnp.float32, precision=HI) + b_ref[...]


def _proj(x, w, b, *, trans=False, nblk=16):
    n = w.shape[0] if trans else w.shape[1]
    bn = n // nblk
    w_spec = (pl.BlockSpec((bn, w.shape[1]), lambda i: (i, 0)) if trans
              else pl.BlockSpec((w.shape[0], bn), lambda i: (0, i)))
    return pl.pallas_call(
        functools.partial(_matmul_body, trans=trans),
        grid=(nblk,),
        in_specs=[
            pl.BlockSpec(x.shape, lambda i: (0, 0)),
            w_spec,
            pl.BlockSpec((bn,), lambda i: (i,)),
        ],
        out_specs=pl.BlockSpec((x.shape[0], bn), lambda i: (0, i)),
        out_shape=jax.ShapeDtypeStruct((x.shape[0], n), jnp.float32),
    )(x, w, b)


def _pred_table_body(e_ref, w1_ref, b1_ref, w2_ref, b2_ref, o_ref):
    h = jnp.maximum(
        lax.dot_general(e_ref[...], w1_ref[...], (((1,), (0,)), ((), ())),
                        preferred_element_type=jnp.float32, precision=HI)
        + b1_ref[...], 0.0)
    o_ref[...] = lax.dot_general(h, w2_ref[...], (((1,), (0,)), ((), ())),
                                 preferred_element_type=jnp.float32,
                                 precision=HI) + b2_ref[...]


def _pred_table(rel_embed_w, w1, b1, w2, b2):
    return pl.pallas_call(
        _pred_table_body,
        out_shape=jax.ShapeDtypeStruct((NREL, D), jnp.float32),
    )(rel_embed_w, w1, b1[None, :], w2, b2[None, :])


def _final_body(wk_ref, wp_ref, vtl_ref, vbl_ref, bv_ref, rr_ref, o_ref):
    fx = (lax.dot_general(wk_ref[...], vtl_ref[...], (((1,), (0,)), ((), ())),
                          preferred_element_type=jnp.float32, precision=HI)
          + lax.dot_general(wp_ref[...], vbl_ref[...], (((1,), (0,)), ((), ())),
                            preferred_element_type=jnp.float32, precision=HI)
          + bv_ref[...])
    rr = rr_ref[...]
    o_ref[...] = jnp.maximum(fx + rr, 0.0) - (fx - rr) ** 2


def _final(wk, wp, vw, vb, rel_rep, nblk=16):
    bn = D // nblk
    return pl.pallas_call(
        _final_body,
        grid=(nblk,),
        in_specs=[
            pl.BlockSpec((Q, D), lambda i: (0, 0)),
            pl.BlockSpec((Q, D), lambda i: (0, 0)),
            pl.BlockSpec((D, bn), lambda i: (0, i)),
            pl.BlockSpec((D, bn), lambda i: (1, i)),
            pl.BlockSpec((bn,), lambda i: (i,)),
            pl.BlockSpec((Q, bn), lambda i: (0, i)),
        ],
        out_specs=pl.BlockSpec((Q, bn), lambda i: (0, i)),
        out_shape=jax.ShapeDtypeStruct((Q, D), jnp.float32),
    )(wk, wp, vw, vw, vb, rel_rep)


_NW = 32
_QPW = Q // _NW
_NCH = BLK // 16
_NEG = float("-inf")
_SCALE = 1.0 / (float(D) ** 0.5)


def _lanes():
    return lax.iota(jnp.int32, 16)


def _merge_top16(pool_v, pool_i, ch, ids):
    c_desc, ci_desc = plsc.sort_key_val(ch, ids, descending=True)
    take = c_desc > pool_v
    nv = jnp.where(take, c_desc, pool_v)
    ni = jnp.where(take, ci_desc, pool_i)
    return plsc.sort_key_val(nv, ni, descending=False)


def _sc_retrieve_body(simb_hbm, bmax_hbm, fbk_hbm, fbv_hbm, ptab_hbm, t_hbm,
                      osim_hbm, ovals_hbm, wk_hbm, wp_hbm,
                      fbval_v, bm_v, surv_v, sidx_v, blk_v, idx8_v, vals8_v,
                      rows_v, prow_v, t_v, wk_v, wp_v, osim_v, ovals_v,
                      sem0, sem1):
    wid = lax.axis_index("s") * 2 + lax.axis_index("c")
    pltpu.sync_copy(fbv_hbm, fbval_v)
    lanes = _lanes()

    def per_query(j, _):
        qg = wid * _QPW + j
        pltpu.sync_copy(bmax_hbm.at[qg], bm_v)

        pool_v = jnp.full((16,), _NEG, jnp.float32)
        pool_i = jnp.zeros((16,), jnp.int32)
        for c in range(_NCH):
            ch = bm_v[pl.ds(c * 16, 16)]
            pool_v, pool_i = _merge_top16(pool_v, pool_i, ch, lanes)
        m8 = jnp.max(jnp.where(lanes == 8, pool_v, _NEG))

        surv_v[pl.ds(0, 16)] = jnp.zeros((16,), jnp.int32)
        cnt = jnp.int32(0)
        for c in range(_NCH):
            ch = bm_v[pl.ds(c * 16, 16)]
            msk = ch >= m8
            pos = cnt + plsc.cumsum(msk.astype(jnp.int32)) - 1
            plsc.store_scatter(surv_v, [pos], lanes + c * 16, mask=msk)
            cnt = cnt + jnp.sum(msk.astype(jnp.int32))
        k = jnp.minimum(cnt, 16)

        sidx_v[...] = surv_v[pl.ds(0, 16)] * Q + qg
        pltpu.async_copy(simb_hbm.at[sidx_v], blk_v, sem0).wait()

        def scan_block(s, carry):
            pool_v, pool_i, thresh = carry
            sb = surv_v[pl.ds(s, 16)][0]
            for c in range(_NCH):
                ch = blk_v[s, pl.ds(c * 16, 16)]
                ids = lanes + (sb * BLK + c * 16)

                def do_merge(args):
                    pv, pi = args
                    pv, pi = _merge_top16(pv, pi, ch, ids)
                    return pv, pi

                pool_v, pool_i = lax.cond(
                    jnp.any(ch > thresh), do_merge, lambda a: a,
                    (pool_v, pool_i))
                thresh = jnp.min(pool_v)
            return pool_v, pool_i, thresh

        pool_v = jnp.full((16,), _NEG, jnp.float32)
        pool_i = jnp.zeros((16,), jnp.int32)
        pool_v, pool_i, _ = lax.fori_loop(
            0, k, scan_block, (pool_v, pool_i, jnp.float32(_NEG)))

        top_v, top_i = plsc.sort_key_val(pool_v, pool_i, descending=True)
        low8 = lanes < 8
        plsc.store_scatter(osim_v, [j * K + lanes], top_v, mask=low8)
        plsc.store_scatter(idx8_v, [lanes], top_i, mask=low8)
        rv = plsc.load_gather(fbval_v, [top_i])
        plsc.store_scatter(ovals_v, [j * K + lanes], rv, mask=low8)
        plsc.store_scatter(vals8_v, [lanes], rv, mask=low8)

        h1 = pltpu.async_copy(fbk_hbm.at[idx8_v], rows_v, sem0)
        h2 = pltpu.async_copy(ptab_hbm.at[vals8_v], prow_v, sem1)
        pltpu.sync_copy(t_hbm.at[qg], t_v)
        h1.wait()
        h2.wait()

        def dot_chunk(i, accs):
            tch = t_v[pl.ds(i * 16, 16)]
            return tuple(accs[kk] + rows_v[kk, pl.ds(i * 16, 16)] * tch
                         for kk in range(K))

        accs = lax.fori_loop(0, D // 16, dot_chunk,
                             tuple(jnp.zeros((16,), jnp.float32)
                                   for _ in range(K)))
        svec = jnp.full((16,), _NEG, jnp.float32)
        for kk in range(K):
            svec = jnp.where(lanes == kk, jnp.sum(accs[kk]), svec)
        mx = jnp.max(svec)
        e = jnp.exp((svec - mx) * _SCALE)
        attn = e / jnp.sum(e)
        aw = [jnp.sum(jnp.where(lanes == kk, attn, 0.0)) for kk in range(K)]

        def wsum_chunk(i, _):
            sl = pl.ds(i * 16, 16)
            acc_k = aw[0] * rows_v[0, sl]
            acc_p = aw[0] * prow_v[0, sl]
            for kk in range(1, K):
                acc_k = acc_k + aw[kk] * rows_v[kk, sl]
                acc_p = acc_p + aw[kk] * prow_v[kk, sl]
            wk_v[sl] = acc_k
            wp_v[sl] = acc_p
            return 0

        lax.fori_loop(0, D // 16, wsum_chunk, 0)
        pltpu.sync_copy(wk_v, wk_hbm.at[qg])
        pltpu.sync_copy(wp_v, wp_hbm.at[qg])
        return 0

    lax.fori_loop(0, _QPW, per_query, 0)
    pltpu.sync_copy(osim_v, osim_hbm.at[pl.ds(wid * _QPW * K, _QPW * K)])
    pltpu.sync_copy(ovals_v, ovals_hbm.at[pl.ds(wid * _QPW * K, _QPW * K)])


def _sc_retrieve(simb, bmax, fb_key, fb_value, ptab, t):
    mesh = plsc.VectorSubcoreMesh(core_axis_name="c", subcore_axis_name="s")
    f = pl.kernel(
        _sc_retrieve_body,
        out_type=[
            jax.ShapeDtypeStruct((Q * K,), jnp.float32),
            jax.ShapeDtypeStruct((Q * K,), jnp.int32),
            jax.ShapeDtypeStruct((Q, D), jnp.float32),
            jax.ShapeDtypeStruct((Q, D), jnp.float32),
        ],
        mesh=mesh,
        compiler_params=pltpu.CompilerParams(needs_layout_passes=False),
        scratch_types=[
            pltpu.VMEM((M,), jnp.int32),
            pltpu.VMEM((NBLK,), jnp.float32),
            pltpu.VMEM((NBLK + 16,), jnp.int32),
            pltpu.VMEM((16,), jnp.int32),
            pltpu.VMEM((16, BLK), jnp.float32),
            pltpu.VMEM((K,), jnp.int32),
            pltpu.VMEM((K,), jnp.int32),
            pltpu.VMEM((K, D), jnp.float32),
            pltpu.VMEM((K, D), jnp.float32),
            pltpu.VMEM((D,), jnp.float32),
            pltpu.VMEM((D,), jnp.float32),
            pltpu.VMEM((D,), jnp.float32),
            pltpu.VMEM((_QPW * K,), jnp.float32),
            pltpu.VMEM((_QPW * K,), jnp.int32),
            pltpu.SemaphoreType.DMA,
            pltpu.SemaphoreType.DMA,
        ],
    )
    return f(simb, bmax, fb_key, fb_value, ptab, t)


def kernel(rel_rep, fb_key, fb_value, rel_embed_w, wpred_w1, wpred_b1,
           wpred_w2, wpred_b2, attn_q_w, attn_q_b, attn_k_w, attn_k_b,
           attn_v_w, attn_v_b):
    q = _proj(rel_rep, attn_q_w, attn_q_b)
    t = _proj(q, attn_k_w, jnp.zeros((D,), jnp.float32), trans=True)
    ptab = _pred_table(rel_embed_w, wpred_w1, wpred_b1, wpred_w2, wpred_b2)

    simb, bmax_bq = _sim_blocked(rel_rep, fb_key)
    bmax = _bmaxT(bmax_bq)

    topk_sim_f, ret_vals_f, wk, wp = _sc_retrieve(
        simb, bmax, fb_key, fb_value, ptab, t)
    topk_sim = topk_sim_f.reshape(Q, K)
    ret_vals = ret_vals_f.reshape(Q, K)

    out = _final(wk, wp, attn_v_w, attn_v_b, rel_rep)
    return out, topk_sim, ret_vals


# speedup vs baseline: 2.6756x; 1.1474x over previous
import functools
import jax
import jax.numpy as jnp
from jax import lax
from jax.experimental import pallas as pl
from jax.experimental.pallas import tpu as pltpu
from jax.experimental.pallas import tpu_sc as plsc

Q = 256
M = 16384
D = 2048
K = 8
NREL = 51
BLK = 128
NBLK = M // BLK
MBLK = 1024
HI = jax.lax.Precision.DEFAULT


def _sim_body(rel_ref, fb_ref, simb_ref, bmax_ref):
    x = rel_ref[...]
    qn = x / (jnp.sqrt(jnp.sum(x * x, axis=-1, keepdims=True)) + 1e-12)
    f = fb_ref[...]
    fn = f / (jnp.sqrt(jnp.sum(f * f, axis=-1, keepdims=True)) + 1e-12)
    s = lax.dot_general(qn, fn, (((1,), (1,)), ((), ())),
                        preferred_element_type=jnp.float32, precision=HI)
    sb = s.reshape(Q, MBLK // BLK, BLK).transpose(1, 0, 2)
    simb_ref[...] = sb.reshape(MBLK // BLK * Q, BLK)
    bmax_ref[...] = jnp.max(sb, axis=2)


def _sim_blocked(rel_rep, fb_key):
    nsteps = M // MBLK
    return pl.pallas_call(
        _sim_body,
        grid=(nsteps,),
        in_specs=[
            pl.BlockSpec((Q, D), lambda i: (0, 0)),
            pl.BlockSpec((MBLK, D), lambda i: (i, 0)),
        ],
        out_specs=[
            pl.BlockSpec((MBLK // BLK * Q, BLK), lambda i: (i, 0)),
            pl.BlockSpec((MBLK // BLK, Q), lambda i: (i, 0)),
        ],
        out_shape=[
            jax.ShapeDtypeStruct((NBLK * Q, BLK), jnp.float32),
            jax.ShapeDtypeStruct((NBLK, Q), jnp.float32),
        ],
    )(rel_rep, fb_key)


def _bmaxT_body(bm_ref, out_ref):
    out_ref[...] = bm_ref[...].T


def _bmaxT(bmax_bq):
    return pl.pallas_call(
        _bmaxT_body,
        out_shape=jax.ShapeDtypeStruct((Q, NBLK), jnp.float32),
    )(bmax_bq)


def _matmul_body(x_ref, w_ref, b_ref, o_ref, *, trans):
    dn = (((1,), (1,)), ((), ())) if trans else (((1,), (0,)), ((), ()))
    o_ref[...] = lax.dot_general(
        x_ref[...], w_ref[...], dn,
        preferred_element_type=jnp.float32, precision=HI) + b_ref[...]


def _proj(x, w, b, *, trans=False, nblk=16):
    n = w.shape[0] if trans else w.shape[1]
    bn = n // nblk
    w_spec = (pl.BlockSpec((bn, w.shape[1]), lambda i: (i, 0)) if trans
              else pl.BlockSpec((w.shape[0], bn), lambda i: (0, i)))
    return pl.pallas_call(
        functools.partial(_matmul_body, trans=trans),
        grid=(nblk,),
        in_specs=[
            pl.BlockSpec(x.shape, lambda i: (0, 0)),
            w_spec,
            pl.BlockSpec((bn,), lambda i: (i,)),
        ],
        out_specs=pl.BlockSpec((x.shape[0], bn), lambda i: (0, i)),
        out_shape=jax.ShapeDtypeStruct((x.shape[0], n), jnp.float32),
    )(x, w, b)


def _pred_table_body(e_ref, w1_ref, b1_ref, w2_ref, b2_ref, o_ref):
    h = jnp.maximum(
        lax.dot_general(e_ref[...], w1_ref[...], (((1,), (0,)), ((), ())),
                        preferred_element_type=jnp.float32, precision=HI)
        + b1_ref[...], 0.0)
    o_ref[...] = lax.dot_general(h, w2_ref[...], (((1,), (0,)), ((), ())),
                                 preferred_element_type=jnp.float32,
                                 precision=HI) + b2_ref[...]


def _pred_table(rel_embed_w, w1, b1, w2, b2):
    return pl.pallas_call(
        _pred_table_body,
        out_shape=jax.ShapeDtypeStruct((NREL, D), jnp.float32),
    )(rel_embed_w, w1, b1[None, :], w2, b2[None, :])


def _final_body(wk_ref, wp_ref, vtl_ref, vbl_ref, bv_ref, rr_ref, o_ref):
    fx = (lax.dot_general(wk_ref[...], vtl_ref[...], (((1,), (0,)), ((), ())),
                          preferred_element_type=jnp.float32, precision=HI)
          + lax.dot_general(wp_ref[...], vbl_ref[...], (((1,), (0,)), ((), ())),
                            preferred_element_type=jnp.float32, precision=HI)
          + bv_ref[...])
    rr = rr_ref[...]
    o_ref[...] = jnp.maximum(fx + rr, 0.0) - (fx - rr) ** 2


def _final(wk, wp, vw, vb, rel_rep, nblk=16):
    bn = D // nblk
    return pl.pallas_call(
        _final_body,
        grid=(nblk,),
        in_specs=[
            pl.BlockSpec((Q, D), lambda i: (0, 0)),
            pl.BlockSpec((Q, D), lambda i: (0, 0)),
            pl.BlockSpec((D, bn), lambda i: (0, i)),
            pl.BlockSpec((D, bn), lambda i: (1, i)),
            pl.BlockSpec((bn,), lambda i: (i,)),
            pl.BlockSpec((Q, bn), lambda i: (0, i)),
        ],
        out_specs=pl.BlockSpec((Q, bn), lambda i: (0, i)),
        out_shape=jax.ShapeDtypeStruct((Q, D), jnp.float32),
    )(wk, wp, vw, vw, vb, rel_rep)


_NW = 32
_QPW = Q // _NW
_NCH = BLK // 16
_SRV = 144
_NEG = float("-inf")
_SCALE = 1.0 / (float(D) ** 0.5)


def _lanes():
    return lax.iota(jnp.int32, 16)


def _merge_top16(pool_v, pool_i, ch, ids):
    c_desc, ci_desc = plsc.sort_key_val(ch, ids, descending=True)
    take = c_desc > pool_v
    nv = jnp.where(take, c_desc, pool_v)
    ni = jnp.where(take, ci_desc, pool_i)
    return plsc.sort_key_val(nv, ni, descending=False)


def _sc_retrieve_body(simb_hbm, bmax_hbm, fbk_hbm, fbv_hbm, ptab_hbm, t_hbm,
                      osim_hbm, ovals_hbm, wk_hbm, wp_hbm,
                      fbval_v, bm8_v, surv_v, sidx_v, sblk_v, idx8_v, vals8_v,
                      rows2_v, prow2_v, t2_v, wk2_v, wp2_v, osim_v, ovals_v,
                      kcnt_s, semA, semB, semC, semD, semW):
    wid = lax.axis_index("s") * 2 + lax.axis_index("c")
    q0 = wid * _QPW
    lanes = _lanes()
    low8 = lanes < 8
    pltpu.sync_copy(fbv_hbm, fbval_v)
    pltpu.sync_copy(bmax_hbm.at[pl.ds(q0, _QPW)], bm8_v)

    def produce(j, _):
        pool_v = jnp.full((16,), _NEG, jnp.float32)
        pool_i = jnp.zeros((16,), jnp.int32)
        for c in range(_NCH):
            ch = bm8_v[j, pl.ds(c * 16, 16)]
            pool_v, pool_i = _merge_top16(pool_v, pool_i, ch, lanes)
        m8 = jnp.max(jnp.where(lanes == 8, pool_v, _NEG))

        off = j * _SRV
        surv_v[pl.ds(off, 16)] = jnp.zeros((16,), jnp.int32)
        cnt = jnp.int32(0)
        for c in range(_NCH):
            ch = bm8_v[j, pl.ds(c * 16, 16)]
            msk = ch >= m8
            pos = off + cnt + plsc.cumsum(msk.astype(jnp.int32)) - 1
            plsc.store_scatter(surv_v, [pos], lanes + c * 16, mask=msk)
            cnt = cnt + jnp.sum(msk.astype(jnp.int32))
        kcnt_s[j] = jnp.minimum(cnt, 16)
        sidx_v[j, pl.ds(0, 16)] = surv_v[pl.ds(off, 16)] * Q + (q0 + j)
        pltpu.async_copy(simb_hbm.at[sidx_v.at[j]], sblk_v.at[j], semA)
        return 0

    lax.fori_loop(0, _QPW, produce, 0)

    def consume(j):
        pltpu.make_async_copy(simb_hbm.at[sidx_v.at[j]], sblk_v.at[j],
                              semA).wait()
        k = kcnt_s[j]

        def scan_block(s, carry):
            pool_v, pool_i, thresh = carry
            sb = surv_v[pl.ds(j * _SRV + s, 16)][0]
            for c in range(_NCH):
                ch = sblk_v[j, s, pl.ds(c * 16, 16)]
                ids = lanes + (sb * BLK + c * 16)

                def do_merge(args):
                    nv, ni = _merge_top16(args[0], args[1], ch, ids)
                    return nv, ni

                pool_v, pool_i = lax.cond(
                    jnp.any(ch > thresh), do_merge, lambda a: a,
                    (pool_v, pool_i))
                thresh = jnp.min(pool_v)
            return pool_v, pool_i, thresh

        pool_v = jnp.full((16,), _NEG, jnp.float32)
        pool_i = jnp.zeros((16,), jnp.int32)
        pool_v, pool_i, _ = lax.fori_loop(
            0, k, scan_block, (pool_v, pool_i, jnp.float32(_NEG)))

        top_v, top_i = plsc.sort_key_val(pool_v, pool_i, descending=True)
        plsc.store_scatter(osim_v, [j * K + lanes], top_v, mask=low8)
        plsc.store_scatter(idx8_v, [j * K + lanes], top_i, mask=low8)
        rv = plsc.load_gather(fbval_v, [top_i])
        plsc.store_scatter(ovals_v, [j * K + lanes], rv, mask=low8)
        plsc.store_scatter(vals8_v, [j * K + lanes], rv, mask=low8)

        jm = lax.rem(j, 2)
        pltpu.async_copy(fbk_hbm.at[idx8_v.at[pl.ds(j * K, K)]],
                         rows2_v.at[jm], semB)
        pltpu.async_copy(ptab_hbm.at[vals8_v.at[pl.ds(j * K, K)]],
                         prow2_v.at[jm], semC)
        pltpu.async_copy(t_hbm.at[q0 + j], t2_v.at[jm], semD)

    def process(x):
        xm = lax.rem(x, 2)
        pltpu.make_async_copy(fbk_hbm.at[idx8_v.at[pl.ds(x * K, K)]],
                              rows2_v.at[xm], semB).wait()
        pltpu.make_async_copy(ptab_hbm.at[vals8_v.at[pl.ds(x * K, K)]],
                              prow2_v.at[xm], semC).wait()
        pltpu.make_async_copy(t_hbm.at[q0 + x], t2_v.at[xm], semD).wait()

        @pl.when(x >= 2)
        def _():
            pltpu.make_async_copy(wk2_v.at[xm], wk_hbm.at[q0 + x - 2],
                                  semW).wait()
            pltpu.make_async_copy(wp2_v.at[xm], wp_hbm.at[q0 + x - 2],
                                  semW).wait()

        def dot_chunk(i, accs):
            tch = t2_v[xm, pl.ds(i * 16, 16)]
            return tuple(accs[kk] + rows2_v[xm, kk, pl.ds(i * 16, 16)] * tch
                         for kk in range(K))

        accs = lax.fori_loop(0, D // 16, dot_chunk,
                             tuple(jnp.zeros((16,), jnp.float32)
                                   for _ in range(K)))
        svec = jnp.full((16,), _NEG, jnp.float32)
        for kk in range(K):
            svec = jnp.where(lanes == kk, jnp.sum(accs[kk]), svec)
        mx = jnp.max(svec)
        e = jnp.exp((svec - mx) * _SCALE)
        attn = e / jnp.sum(e)
        aw = [jnp.sum(jnp.where(lanes == kk, attn, 0.0)) for kk in range(K)]

        def wsum_chunk(i, _):
            sl = pl.ds(i * 16, 16)
            acc_k = aw[0] * rows2_v[xm, 0, sl]
            acc_p = aw[0] * prow2_v[xm, 0, sl]
            for kk in range(1, K):
                acc_k = acc_k + aw[kk] * rows2_v[xm, kk, sl]
                acc_p = acc_p + aw[kk] * prow2_v[xm, kk, sl]
            wk2_v[xm, sl] = acc_k
            wp2_v[xm, sl] = acc_p
            return 0

        lax.fori_loop(0, D // 16, wsum_chunk, 0)
        pltpu.async_copy(wk2_v.at[xm], wk_hbm.at[q0 + x], semW)
        pltpu.async_copy(wp2_v.at[xm], wp_hbm.at[q0 + x], semW)

    def step(i, _):
        @pl.when(i < _QPW)
        def _():
            consume(i)

        @pl.when(i >= 1)
        def _():
            process(i - 1)
        return 0

    lax.fori_loop(0, _QPW + 1, step, 0)

    for x in (_QPW - 2, _QPW - 1):
        xm = x % 2
        pltpu.make_async_copy(wk2_v.at[xm], wk_hbm.at[q0 + x], semW).wait()
        pltpu.make_async_copy(wp2_v.at[xm], wp_hbm.at[q0 + x], semW).wait()

    pltpu.sync_copy(osim_v, osim_hbm.at[pl.ds(q0 * K, _QPW * K)])
    pltpu.sync_copy(ovals_v, ovals_hbm.at[pl.ds(q0 * K, _QPW * K)])


def _sc_retrieve(simb, bmax, fb_key, fb_value, ptab, t):
    mesh = plsc.VectorSubcoreMesh(core_axis_name="c", subcore_axis_name="s")
    f = pl.kernel(
        _sc_retrieve_body,
        out_type=[
            jax.ShapeDtypeStruct((Q * K,), jnp.float32),
            jax.ShapeDtypeStruct((Q * K,), jnp.int32),
            jax.ShapeDtypeStruct((Q, D), jnp.float32),
            jax.ShapeDtypeStruct((Q, D), jnp.float32),
        ],
        mesh=mesh,
        compiler_params=pltpu.CompilerParams(needs_layout_passes=False),
        scratch_types=[
            pltpu.VMEM((M,), jnp.int32),
            pltpu.VMEM((_QPW, NBLK), jnp.float32),
            pltpu.VMEM((_QPW * _SRV,), jnp.int32),
            pltpu.VMEM((_QPW, 16), jnp.int32),
            pltpu.VMEM((_QPW, 16, BLK), jnp.float32),
            pltpu.VMEM((_QPW * K,), jnp.int32),
            pltpu.VMEM((_QPW * K,), jnp.int32),
            pltpu.VMEM((2, K, D), jnp.float32),
            pltpu.VMEM((2, K, D), jnp.float32),
            pltpu.VMEM((2, D), jnp.float32),
            pltpu.VMEM((2, D), jnp.float32),
            pltpu.VMEM((2, D), jnp.float32),
            pltpu.VMEM((_QPW * K,), jnp.float32),
            pltpu.VMEM((_QPW * K,), jnp.int32),
            pltpu.SMEM((_QPW,), jnp.int32),
            pltpu.SemaphoreType.DMA,
            pltpu.SemaphoreType.DMA,
            pltpu.SemaphoreType.DMA,
            pltpu.SemaphoreType.DMA,
            pltpu.SemaphoreType.DMA,
        ],
    )
    return f(simb, bmax, fb_key, fb_value, ptab, t)


def kernel(rel_rep, fb_key, fb_value, rel_embed_w, wpred_w1, wpred_b1,
           wpred_w2, wpred_b2, attn_q_w, attn_q_b, attn_k_w, attn_k_b,
           attn_v_w, attn_v_b):
    q = _proj(rel_rep, attn_q_w, attn_q_b)
    t = _proj(q, attn_k_w, jnp.zeros((D,), jnp.float32), trans=True)
    ptab = _pred_table(rel_embed_w, wpred_w1, wpred_b1, wpred_w2, wpred_b2)

    simb, bmax_bq = _sim_blocked(rel_rep, fb_key)
    bmax = _bmaxT(bmax_bq)

    topk_sim_f, ret_vals_f, wk, wp = _sc_retrieve(
        simb, bmax, fb_key, fb_value, ptab, t)
    topk_sim = topk_sim_f.reshape(Q, K)
    ret_vals = ret_vals_f.reshape(Q, K)

    out = _final(wk, wp, attn_v_w, attn_v_b, rel_rep)
    return out, topk_sim, ret_vals


# speedup vs baseline: 2.8073x; 1.0492x over previous
import functools
import jax
import jax.numpy as jnp
from jax import lax
from jax.experimental import pallas as pl
from jax.experimental.pallas import tpu as pltpu
from jax.experimental.pallas import tpu_sc as plsc

Q = 256
M = 16384
D = 2048
K = 8
NREL = 51
BLK = 128
NBLK = M // BLK
MBLK = 1024
HI = jax.lax.Precision.DEFAULT


def _sim_body(rel_ref, fb_ref, simb_ref, bmax_ref):
    x = rel_ref[...]
    qn = x / (jnp.sqrt(jnp.sum(x * x, axis=-1, keepdims=True)) + 1e-12)
    f = fb_ref[...]
    fn = f / (jnp.sqrt(jnp.sum(f * f, axis=-1, keepdims=True)) + 1e-12)
    s = lax.dot_general(qn, fn, (((1,), (1,)), ((), ())),
                        preferred_element_type=jnp.float32, precision=HI)
    sb = s.reshape(Q, MBLK // BLK, BLK).transpose(1, 0, 2)
    simb_ref[...] = sb.reshape(MBLK // BLK * Q, BLK)
    bmax_ref[...] = jnp.max(sb, axis=2)


def _sim_blocked(rel_rep, fb_key):
    nsteps = M // MBLK
    return pl.pallas_call(
        _sim_body,
        grid=(nsteps,),
        in_specs=[
            pl.BlockSpec((Q, D), lambda i: (0, 0)),
            pl.BlockSpec((MBLK, D), lambda i: (i, 0)),
        ],
        out_specs=[
            pl.BlockSpec((MBLK // BLK * Q, BLK), lambda i: (i, 0)),
            pl.BlockSpec((MBLK // BLK, Q), lambda i: (i, 0)),
        ],
        out_shape=[
            jax.ShapeDtypeStruct((NBLK * Q, BLK), jnp.float32),
            jax.ShapeDtypeStruct((NBLK, Q), jnp.float32),
        ],
    )(rel_rep, fb_key)


def _bmaxT_body(bm_ref, out_ref):
    out_ref[...] = bm_ref[...].T


def _bmaxT(bmax_bq):
    return pl.pallas_call(
        _bmaxT_body,
        out_shape=jax.ShapeDtypeStruct((Q, NBLK), jnp.float32),
    )(bmax_bq)


def _qt_body(rel_ref, wq_ref, bq_ref, wk_ref, t_ref):
    qb = lax.dot_general(rel_ref[...], wq_ref[...], (((1,), (0,)), ((), ())),
                         preferred_element_type=jnp.float32,
                         precision=HI) + bq_ref[...]
    contrib = lax.dot_general(qb, wk_ref[...], (((1,), (1,)), ((), ())),
                              preferred_element_type=jnp.float32, precision=HI)

    @pl.when(pl.program_id(0) == 0)
    def _():
        t_ref[...] = contrib

    @pl.when(pl.program_id(0) > 0)
    def _():
        t_ref[...] = t_ref[...] + contrib


def _qt(rel_rep, wq, bq, wk, nblk=16):
    cb = D // nblk
    return pl.pallas_call(
        _qt_body,
        grid=(nblk,),
        in_specs=[
            pl.BlockSpec((Q, D), lambda i: (0, 0)),
            pl.BlockSpec((D, cb), lambda i: (0, i)),
            pl.BlockSpec((cb,), lambda i: (i,)),
            pl.BlockSpec((D, cb), lambda i: (0, i)),
        ],
        out_specs=pl.BlockSpec((Q, D), lambda i: (0, 0)),
        out_shape=jax.ShapeDtypeStruct((Q, D), jnp.float32),
    )(rel_rep, wq, bq, wk)


def _ptab2_body(e_ref, w1_ref, b1_ref, w2_ref, b2_ref, v_ref, o_ref, ptab_scr):
    @pl.when(pl.program_id(0) == 0)
    def _():
        h = jnp.maximum(
            lax.dot_general(e_ref[...], w1_ref[...], (((1,), (0,)), ((), ())),
                            preferred_element_type=jnp.float32, precision=HI)
            + b1_ref[...], 0.0)
        ptab_scr[...] = lax.dot_general(
            h, w2_ref[...], (((1,), (0,)), ((), ())),
            preferred_element_type=jnp.float32, precision=HI) + b2_ref[...]

    o_ref[...] = lax.dot_general(
        ptab_scr[...], v_ref[...], (((1,), (0,)), ((), ())),
        preferred_element_type=jnp.float32, precision=HI)


def _ptab2(rel_embed_w, w1, b1, w2, b2, vw, nblk=4):
    cb = D // nblk
    ed = rel_embed_w.shape[1]
    hd = w1.shape[1]
    return pl.pallas_call(
        _ptab2_body,
        grid=(nblk,),
        in_specs=[
            pl.BlockSpec((NREL, ed), lambda i: (0, 0)),
            pl.BlockSpec((ed, hd), lambda i: (0, 0)),
            pl.BlockSpec((1, hd), lambda i: (0, 0)),
            pl.BlockSpec((hd, D), lambda i: (0, 0)),
            pl.BlockSpec((1, D), lambda i: (0, 0)),
            pl.BlockSpec((D, cb), lambda i: (1, i)),
        ],
        out_specs=pl.BlockSpec((NREL, cb), lambda i: (0, i)),
        out_shape=jax.ShapeDtypeStruct((NREL, D), jnp.float32),
        scratch_shapes=[pltpu.VMEM((NREL, D), jnp.float32)],
    )(rel_embed_w, w1, b1[None, :], w2, b2[None, :], vw)


def _final_body(wk_ref, attn_ref, vals_ref, p2_ref, vtl_ref, bv_ref, rr_ref,
                o_ref):
    fx = lax.dot_general(wk_ref[...], vtl_ref[...], (((1,), (0,)), ((), ())),
                         preferred_element_type=jnp.float32, precision=HI)
    rids = lax.broadcasted_iota(jnp.int32, (Q, NREL), 1)
    attn = attn_ref[...]
    vals = vals_ref[...]
    wgt = jnp.zeros((Q, NREL), jnp.float32)
    for kk in range(K):
        wgt = wgt + attn[:, kk:kk + 1] * (vals[:, kk:kk + 1] == rids)
    fx = fx + lax.dot_general(wgt, p2_ref[...], (((1,), (0,)), ((), ())),
                              preferred_element_type=jnp.float32,
                              precision=HI) + bv_ref[...]
    rr = rr_ref[...]
    o_ref[...] = jnp.maximum(fx + rr, 0.0) - (fx - rr) ** 2


def _final(wk, attn, vals, ptab2, vw, vb, rel_rep, nblk=16):
    bn = D // nblk
    return pl.pallas_call(
        _final_body,
        grid=(nblk,),
        in_specs=[
            pl.BlockSpec((Q, D), lambda i: (0, 0)),
            pl.BlockSpec((Q, K), lambda i: (0, 0)),
            pl.BlockSpec((Q, K), lambda i: (0, 0)),
            pl.BlockSpec((NREL, bn), lambda i: (0, i)),
            pl.BlockSpec((D, bn), lambda i: (0, i)),
            pl.BlockSpec((bn,), lambda i: (i,)),
            pl.BlockSpec((Q, bn), lambda i: (0, i)),
        ],
        out_specs=pl.BlockSpec((Q, bn), lambda i: (0, i)),
        out_shape=jax.ShapeDtypeStruct((Q, D), jnp.float32),
    )(wk, attn, vals, ptab2, vw, vb, rel_rep)


_NW = 32
_QPW = Q // _NW
_NCH = BLK // 16
_SRV = 144
_NEG = float("-inf")
_SCALE = 1.0 / (float(D) ** 0.5)


def _lanes():
    return lax.iota(jnp.int32, 16)


def _merge_top16(pool_v, pool_i, ch, ids):
    c_desc, ci_desc = plsc.sort_key_val(ch, ids, descending=True)
    take = c_desc > pool_v
    nv = jnp.where(take, c_desc, pool_v)
    ni = jnp.where(take, ci_desc, pool_i)
    return plsc.sort_key_val(nv, ni, descending=False)


def _sc_retrieve_body(simb_hbm, bmax_hbm, fbk_hbm, fbv_hbm, t_hbm,
                      osim_hbm, ovals_hbm, oattn_hbm, wk_hbm,
                      fbval_v, bm8_v, surv_v, sidx_v, sblk_v, idx8_v,
                      rows2_v, t2_v, wk2_v, osim_v, ovals_v, oattn_v,
                      kcnt_s, semA, semB, semD, semW):
    wid = lax.axis_index("s") * 2 + lax.axis_index("c")
    q0 = wid * _QPW
    lanes = _lanes()
    low8 = lanes < 8
    pltpu.sync_copy(fbv_hbm, fbval_v)
    pltpu.sync_copy(bmax_hbm.at[pl.ds(q0, _QPW)], bm8_v)

    def produce(j, _):
        pool_v = jnp.full((16,), _NEG, jnp.float32)
        pool_i = jnp.zeros((16,), jnp.int32)
        for c in range(_NCH):
            ch = bm8_v[j, pl.ds(c * 16, 16)]
            pool_v, pool_i = _merge_top16(pool_v, pool_i, ch, lanes)
        m8 = jnp.max(jnp.where(lanes == 8, pool_v, _NEG))

        off = j * _SRV
        surv_v[pl.ds(off, 16)] = jnp.zeros((16,), jnp.int32)
        cnt = jnp.int32(0)
        for c in range(_NCH):
            ch = bm8_v[j, pl.ds(c * 16, 16)]
            msk = ch >= m8
            pos = off + cnt + plsc.cumsum(msk.astype(jnp.int32)) - 1
            plsc.store_scatter(surv_v, [pos], lanes + c * 16, mask=msk)
            cnt = cnt + jnp.sum(msk.astype(jnp.int32))
        kcnt_s[j] = jnp.minimum(cnt, 16)
        sidx_v[j, pl.ds(0, 16)] = surv_v[pl.ds(off, 16)] * Q + (q0 + j)
        pltpu.async_copy(simb_hbm.at[sidx_v.at[j]], sblk_v.at[j], semA)
        return 0

    lax.fori_loop(0, _QPW, produce, 0)

    def consume(j):
        pltpu.make_async_copy(simb_hbm.at[sidx_v.at[j]], sblk_v.at[j],
                              semA).wait()
        k = kcnt_s[j]

        def scan_block(s, carry):
            pool_v, pool_i, thresh = carry
            sb = surv_v[pl.ds(j * _SRV + s, 16)][0]
            for c in range(_NCH):
                ch = sblk_v[j, s, pl.ds(c * 16, 16)]
                ids = lanes + (sb * BLK + c * 16)

                def do_merge(args):
                    nv, ni = _merge_top16(args[0], args[1], ch, ids)
                    return nv, ni

                pool_v, pool_i = lax.cond(
                    jnp.any(ch > thresh), do_merge, lambda a: a,
                    (pool_v, pool_i))
                thresh = jnp.min(pool_v)
            return pool_v, pool_i, thresh

        pool_v = jnp.full((16,), _NEG, jnp.float32)
        pool_i = jnp.zeros((16,), jnp.int32)
        pool_v, pool_i, _ = lax.fori_loop(
            0, k, scan_block, (pool_v, pool_i, jnp.float32(_NEG)))

        top_v, top_i = plsc.sort_key_val(pool_v, pool_i, descending=True)
        plsc.store_scatter(osim_v, [j * K + lanes], top_v, mask=low8)
        plsc.store_scatter(idx8_v, [j * K + lanes], top_i, mask=low8)
        rv = plsc.load_gather(fbval_v, [top_i])
        plsc.store_scatter(ovals_v, [j * K + lanes], rv, mask=low8)

        jm = lax.rem(j, 2)
        pltpu.async_copy(fbk_hbm.at[idx8_v.at[pl.ds(j * K, K)]],
                         rows2_v.at[jm], semB)
        pltpu.async_copy(t_hbm.at[q0 + j], t2_v.at[jm], semD)

    def process(x):
        xm = lax.rem(x, 2)
        pltpu.make_async_copy(fbk_hbm.at[idx8_v.at[pl.ds(x * K, K)]],
                              rows2_v.at[xm], semB).wait()
        pltpu.make_async_copy(t_hbm.at[q0 + x], t2_v.at[xm], semD).wait()

        @pl.when(x >= 2)
        def _():
            pltpu.make_async_copy(wk2_v.at[xm], wk_hbm.at[q0 + x - 2],
                                  semW).wait()

        def dot_chunk(i, accs):
            tch = t2_v[xm, pl.ds(i * 16, 16)]
            return tuple(accs[kk] + rows2_v[xm, kk, pl.ds(i * 16, 16)] * tch
                         for kk in range(K))

        accs = lax.fori_loop(0, D // 16, dot_chunk,
                             tuple(jnp.zeros((16,), jnp.float32)
                                   for _ in range(K)))
        svec = jnp.full((16,), _NEG, jnp.float32)
        for kk in range(K):
            svec = jnp.where(lanes == kk, jnp.sum(accs[kk]), svec)
        mx = jnp.max(svec)
        e = jnp.exp((svec - mx) * _SCALE)
        attn = e / jnp.sum(e)
        plsc.store_scatter(oattn_v, [x * K + _lanes()], attn, mask=_lanes() < 8)
        aw = [jnp.sum(jnp.where(lanes == kk, attn, 0.0)) for kk in range(K)]

        def wsum_chunk(i, _):
            sl = pl.ds(i * 16, 16)
            acc_k = aw[0] * rows2_v[xm, 0, sl]
            for kk in range(1, K):
                acc_k = acc_k + aw[kk] * rows2_v[xm, kk, sl]
            wk2_v[xm, sl] = acc_k
            return 0

        lax.fori_loop(0, D // 16, wsum_chunk, 0)
        pltpu.async_copy(wk2_v.at[xm], wk_hbm.at[q0 + x], semW)

    def step(i, _):
        @pl.when(i < _QPW)
        def _():
            consume(i)

        @pl.when(i >= 1)
        def _():
            process(i - 1)
        return 0

    lax.fori_loop(0, _QPW + 1, step, 0)

    for x in (_QPW - 2, _QPW - 1):
        xm = x % 2
        pltpu.make_async_copy(wk2_v.at[xm], wk_hbm.at[q0 + x], semW).wait()

    pltpu.sync_copy(osim_v, osim_hbm.at[pl.ds(q0 * K, _QPW * K)])
    pltpu.sync_copy(ovals_v, ovals_hbm.at[pl.ds(q0 * K, _QPW * K)])
    pltpu.sync_copy(oattn_v, oattn_hbm.at[pl.ds(q0 * K, _QPW * K)])


def _sc_retrieve(simb, bmax, fb_key, fb_value, t):
    mesh = plsc.VectorSubcoreMesh(core_axis_name="c", subcore_axis_name="s")
    f = pl.kernel(
        _sc_retrieve_body,
        out_type=[
            jax.ShapeDtypeStruct((Q * K,), jnp.float32),
            jax.ShapeDtypeStruct((Q * K,), jnp.int32),
            jax.ShapeDtypeStruct((Q * K,), jnp.float32),
            jax.ShapeDtypeStruct((Q, D), jnp.float32),
        ],
        mesh=mesh,
        compiler_params=pltpu.CompilerParams(needs_layout_passes=False),
        scratch_types=[
            pltpu.VMEM((M,), jnp.int32),
            pltpu.VMEM((_QPW, NBLK), jnp.float32),
            pltpu.VMEM((_QPW * _SRV,), jnp.int32),
            pltpu.VMEM((_QPW, 16), jnp.int32),
            pltpu.VMEM((_QPW, 16, BLK), jnp.float32),
            pltpu.VMEM((_QPW * K,), jnp.int32),
            pltpu.VMEM((2, K, D), jnp.float32),
            pltpu.VMEM((2, D), jnp.float32),
            pltpu.VMEM((2, D), jnp.float32),
            pltpu.VMEM((_QPW * K,), jnp.float32),
            pltpu.VMEM((_QPW * K,), jnp.int32),
            pltpu.VMEM((_QPW * K,), jnp.float32),
            pltpu.SMEM((_QPW,), jnp.int32),
            pltpu.SemaphoreType.DMA,
            pltpu.SemaphoreType.DMA,
            pltpu.SemaphoreType.DMA,
            pltpu.SemaphoreType.DMA,
        ],
    )
    return f(simb, bmax, fb_key, fb_value, t)


def kernel(rel_rep, fb_key, fb_value, rel_embed_w, wpred_w1, wpred_b1,
           wpred_w2, wpred_b2, attn_q_w, attn_q_b, attn_k_w, attn_k_b,
           attn_v_w, attn_v_b):
    t = _qt(rel_rep, attn_q_w, attn_q_b, attn_k_w)
    simb, bmax_bq = _sim_blocked(rel_rep, fb_key)
    bmax = _bmaxT(bmax_bq)

    topk_sim_f, ret_vals_f, attn_f, wk = _sc_retrieve(
        simb, bmax, fb_key, fb_value, t)
    topk_sim = topk_sim_f.reshape(Q, K)
    ret_vals = ret_vals_f.reshape(Q, K)
    attn = attn_f.reshape(Q, K)

    ptab2 = _ptab2(rel_embed_w, wpred_w1, wpred_b1, wpred_w2, wpred_b2,
                   attn_v_w)
    out = _final(wk, attn, ret_vals, ptab2, attn_v_w, attn_v_b, rel_rep)
    return out, topk_sim, ret_vals
